# Initial kernel scaffold; baseline (speedup 1.0000x reference)
import jax
import jax.numpy as jnp
from jax import lax
from jax.experimental import pallas as pl
from jax.experimental.pallas import tpu as pltpu
from jax.experimental.pallas import tpu_sc as plsc

N = 10000
NP = 10240
E = 320000
H = 128
L = 4

NC = 2
NS = 16
NW = NC * NS
EPT = E // NW
KB = 80
NB = EPT // KB
SEG = NP // NS
GB = NP // (KB * NW)

_F32 = jnp.float32


def _sc_mesh():
  return plsc.VectorSubcoreMesh(core_axis_name="c", subcore_axis_name="s")


def _sc_deg_emb_body(src_h, dst_h, hpad_h, emb_h, deg_h, x0_h,
                     deg_v, idx_v, rows_v, sem):
  c = lax.axis_index("c")
  s = lax.axis_index("s")
  wid = c * NS + s

  def zero(i, carry):
    deg_v[0, pl.ds(i * 16, 16)] = jnp.zeros((16,), _F32)
    deg_v[1, pl.ds(i * 16, 16)] = jnp.zeros((16,), _F32)
    return carry
  lax.fori_loop(0, NP // 16, zero, 0)

  eb = wid * EPT
  ones = jnp.ones((16,), _F32)

  def edge_batch(b, carry):
    base = eb + b * KB
    pltpu.sync_copy(src_h.at[pl.ds(base, KB)], idx_v)

    def upd_src(k, carry2):
      plsc.addupdate_scatter(deg_v.at[0], [idx_v[pl.ds(k * 16, 16)]], ones)
      return carry2
    lax.fori_loop(0, KB // 16, upd_src, 0)

    pltpu.sync_copy(dst_h.at[pl.ds(base, KB)], idx_v)

    def upd_dst(k, carry2):
      plsc.addupdate_scatter(deg_v.at[1], [idx_v[pl.ds(k * 16, 16)]], ones)
      return carry2
    lax.fori_loop(0, KB // 16, upd_dst, 0)
    return carry
  lax.fori_loop(0, NB, edge_batch, 0)

  pltpu.sync_copy(deg_v, deg_h.at[wid])

  for t in range(GB):
    j = wid + t * NW
    pltpu.sync_copy(hpad_h.at[pl.ds(j * KB, KB)], idx_v)
    pltpu.async_copy(emb_h.at[idx_v], rows_v, sem).wait()
    pltpu.sync_copy(rows_v, x0_h.at[pl.ds(j * KB, KB)])


def _sc_deg_emb(src, dst, hpad, emb):
  fn = pl.kernel(
      _sc_deg_emb_body,
      out_type=[
          jax.ShapeDtypeStruct((NW, 2, NP), _F32),
          jax.ShapeDtypeStruct((NP, H), _F32),
      ],
      mesh=_sc_mesh(),
      scratch_types=[
          pltpu.VMEM((2, NP), _F32),
          pltpu.VMEM((KB,), jnp.int32),
          pltpu.VMEM((KB, H), _F32),
          pltpu.SemaphoreType.DMA,
      ],
  )
  return fn(src, dst, hpad, emb)


def _sc_scatter_body(xs_h, src_h, dst_h, zz_h, out_h,
                     agg_sh, sidx_v, didx_v, rows_v, sem):
  c = lax.axis_index("c")
  s = lax.axis_index("s")

  pltpu.sync_copy(zz_h, agg_sh.at[pl.ds(s * SEG, SEG)])
  plsc.subcore_barrier()

  eb = (c * NS + s) * EPT

  def edge_batch(b, carry):
    base = eb + b * KB
    pltpu.sync_copy(src_h.at[pl.ds(base, KB)], sidx_v)
    pltpu.sync_copy(dst_h.at[pl.ds(base, KB)], didx_v)
    pltpu.async_copy(xs_h.at[sidx_v], rows_v, sem).wait()
    pltpu.sync_copy(rows_v, agg_sh.at[didx_v], add=True)
    return carry
  lax.fori_loop(0, NB, edge_batch, 0)

  plsc.subcore_barrier()
  pltpu.sync_copy(agg_sh.at[pl.ds(s * SEG, SEG)],
                  out_h.at[c, pl.ds(s * SEG, SEG)])


def _sc_scatter(xs, src, dst, zz):
  fn = pl.kernel(
      _sc_scatter_body,
      out_type=jax.ShapeDtypeStruct((NC, NP, H), _F32),
      mesh=_sc_mesh(),
      scratch_types=[
          pltpu.VMEM_SHARED((NP, H), _F32),
          pltpu.VMEM((KB,), jnp.int32),
          pltpu.VMEM((KB,), jnp.int32),
          pltpu.VMEM((KB, H), _F32),
          pltpu.SemaphoreType.DMA,
      ],
  )
  return fn(xs, src, dst, zz)


def _prep_body(dpt_ref, x0_ref, norms_ref, xs_ref):
  d = jnp.sum(dpt_ref[...], axis=-1)
  nrm = jnp.where(d > 0, lax.rsqrt(jnp.maximum(d, 1.0)), 0.0)
  norms_ref[...] = nrm
  xs_ref[...] = x0_ref[...] * nrm[:, 0:1]


def _prep(dpt, x0):
  bm = 2048
  return pl.pallas_call(
      _prep_body,
      grid=(NP // bm,),
      in_specs=[
          pl.BlockSpec((bm, 2, NW), lambda i: (i, 0, 0)),
          pl.BlockSpec((bm, H), lambda i: (i, 0)),
      ],
      out_specs=[
          pl.BlockSpec((bm, 2), lambda i: (i, 0)),
          pl.BlockSpec((bm, H), lambda i: (i, 0)),
      ],
      out_shape=[
          jax.ShapeDtypeStruct((NP, 2), _F32),
          jax.ShapeDtypeStruct((NP, H), _F32),
      ],
  )(dpt, x0)


def _layer_body(agg0_ref, agg1_ref, norms_ref, x_ref, w_ref, b_ref,
                g_ref, be_ref, xo_ref, xso_ref):
  agg = (agg0_ref[...] + agg1_ref[...]) * norms_ref[:, 1:2]
  y = jnp.dot(agg, w_ref[...], preferred_element_type=_F32) + b_ref[...]
  xo = x_ref[...] + jnp.maximum(g_ref[...] * y + be_ref[...], 0.0)
  xo_ref[...] = xo
  xso_ref[...] = xo * norms_ref[:, 0:1]


def _layer(agg0, agg1, norms, x, w, b, g, be):
  bm = 1024
  return pl.pallas_call(
      _layer_body,
      grid=(NP // bm,),
      in_specs=[
          pl.BlockSpec((bm, H), lambda i: (i, 0)),
          pl.BlockSpec((bm, H), lambda i: (i, 0)),
          pl.BlockSpec((bm, 2), lambda i: (i, 0)),
          pl.BlockSpec((bm, H), lambda i: (i, 0)),
          pl.BlockSpec((H, H), lambda i: (0, 0)),
          pl.BlockSpec((1, H), lambda i: (0, 0)),
          pl.BlockSpec((1, H), lambda i: (0, 0)),
          pl.BlockSpec((1, H), lambda i: (0, 0)),
      ],
      out_specs=[
          pl.BlockSpec((bm, H), lambda i: (i, 0)),
          pl.BlockSpec((bm, H), lambda i: (i, 0)),
      ],
      out_shape=[
          jax.ShapeDtypeStruct((NP, H), _F32),
          jax.ShapeDtypeStruct((NP, H), _F32),
      ],
  )(agg0, agg1, norms, x, w, b, g, be)


def _readout_body(x_ref, w1_ref, b1_ref, w2_ref, b2_ref, w3_ref, b3_ref,
                  out_ref, acc_ref):
  i = pl.program_id(0)

  @pl.when(i == 0)
  def _():
    acc_ref[...] = jnp.zeros_like(acc_ref)

  acc_ref[...] += jnp.sum(x_ref[...], axis=0, keepdims=True)

  @pl.when(i == pl.num_programs(0) - 1)
  def _():
    hg = acc_ref[...] * (1.0 / N)
    y = jnp.dot(hg, w1_ref[...], preferred_element_type=_F32) + b1_ref[...]
    y = jnp.maximum(y, 0.0)
    y = jnp.dot(y, w2_ref[...], preferred_element_type=_F32) + b2_ref[...]
    y = jnp.maximum(y, 0.0)
    out_ref[...] = (jnp.dot(y, w3_ref[...], preferred_element_type=_F32)
                    + b3_ref[...])


def _readout(x, w1, b1, w2, b2, w3, b3):
  bm = 400
  return pl.pallas_call(
      _readout_body,
      grid=(N // bm,),
      in_specs=[
          pl.BlockSpec((bm, H), lambda i: (i, 0)),
          pl.BlockSpec((H, H // 2), lambda i: (0, 0)),
          pl.BlockSpec((1, H // 2), lambda i: (0, 0)),
          pl.BlockSpec((H // 2, H // 4), lambda i: (0, 0)),
          pl.BlockSpec((1, H // 4), lambda i: (0, 0)),
          pl.BlockSpec((H // 4, 1), lambda i: (0, 0)),
          pl.BlockSpec((1, 1), lambda i: (0, 0)),
      ],
      out_specs=pl.BlockSpec((1, 1), lambda i: (0, 0)),
      out_shape=jax.ShapeDtypeStruct((1, 1), _F32),
      scratch_shapes=[pltpu.VMEM((1, H), _F32)],
  )(x, w1, b1, w2, b2, w3, b3)


def kernel(h, edge_index, e, emb, Ws, bs, gammas, betas,
           mlpW1, mlpb1, mlpW2, mlpb2, mlpW3, mlpb3):
  src = edge_index[0]
  dst = edge_index[1]
  hpad = jnp.concatenate([h, jnp.zeros((NP - N,), jnp.int32)])
  zz = jnp.zeros((SEG, H), _F32)

  deg_part, x0 = _sc_deg_emb(src, dst, hpad, emb)
  dpt = jnp.transpose(deg_part, (2, 1, 0))
  norms, xs = _prep(dpt, x0)

  x = x0
  for i in range(L):
    aggs = _sc_scatter(xs, src, dst, zz)
    x, xs = _layer(aggs[0], aggs[1], norms, x,
                   Ws[i], bs[i].reshape(1, H),
                   gammas[i].reshape(1, H), betas[i].reshape(1, H))

  return _readout(x, mlpW1, mlpb1.reshape(1, H // 2),
                  mlpW2, mlpb2.reshape(1, H // 4),
                  mlpW3, mlpb3.reshape(1, 1))


# speedup vs baseline: 4.1614x; 4.1614x over previous
import jax
import jax.numpy as jnp
from jax import lax
from jax.experimental import pallas as pl
from jax.experimental.pallas import tpu as pltpu
from jax.experimental.pallas import tpu_sc as plsc

N = 10000
NP = 10240
E = 320000
H = 128
L = 4

NC = 2
NS = 16
NW = NC * NS
EPT = E // NW
KB = 80
NB = EPT // KB
SEG = NP // NS
GB = NP // (KB * NW)

_F32 = jnp.float32


def _sc_mesh():
  return plsc.VectorSubcoreMesh(core_axis_name="c", subcore_axis_name="s",
                                num_cores=NC, num_subcores=NS)


def _sc_deg_emb_body(src_h, dst_h, hpad_h, emb_h, zv_h, deg_h, x0_h,
                     dego_sh, degi_sh, ones_v, idx_v, rows_v, sem):
  c = lax.axis_index("c")
  s = lax.axis_index("s")
  wid = c * NS + s

  pltpu.sync_copy(zv_h, dego_sh.at[pl.ds(s * SEG, SEG)])
  pltpu.sync_copy(zv_h, degi_sh.at[pl.ds(s * SEG, SEG)])

  def fill_ones(i, carry):
    ones_v[pl.ds(i * 16, 16)] = jnp.ones((16,), _F32)
    return carry
  lax.fori_loop(0, KB // 16, fill_ones, 0)
  plsc.subcore_barrier()

  eb = (c * NS + s) * EPT

  def edge_batch(b, carry):
    base = eb + b * KB
    pltpu.sync_copy(src_h.at[pl.ds(base, KB)], idx_v)
    pltpu.sync_copy(ones_v, dego_sh.at[idx_v], add=True)
    pltpu.sync_copy(dst_h.at[pl.ds(base, KB)], idx_v)
    pltpu.sync_copy(ones_v, degi_sh.at[idx_v], add=True)
    return carry
  lax.fori_loop(0, NB, edge_batch, 0)

  plsc.subcore_barrier()
  pltpu.sync_copy(dego_sh.at[pl.ds(s * SEG, SEG)],
                  deg_h.at[c, 0, pl.ds(s * SEG, SEG)])
  pltpu.sync_copy(degi_sh.at[pl.ds(s * SEG, SEG)],
                  deg_h.at[c, 1, pl.ds(s * SEG, SEG)])

  for t in range(GB):
    j = wid + t * NW
    pltpu.sync_copy(hpad_h.at[pl.ds(j * KB, KB)], idx_v)
    pltpu.async_copy(emb_h.at[idx_v], rows_v, sem).wait()
    pltpu.sync_copy(rows_v, x0_h.at[pl.ds(j * KB, KB)])


def _sc_deg_emb(src, dst, hpad, emb, zv):
  fn = pl.kernel(
      _sc_deg_emb_body,
      out_type=[
          jax.ShapeDtypeStruct((NC, 2, NP), _F32),
          jax.ShapeDtypeStruct((NP, H), _F32),
      ],
      mesh=_sc_mesh(),
      scratch_types=[
          pltpu.VMEM_SHARED((NP,), _F32),
          pltpu.VMEM_SHARED((NP,), _F32),
          pltpu.VMEM((KB,), _F32),
          pltpu.VMEM((KB,), jnp.int32),
          pltpu.VMEM((KB, H), _F32),
          pltpu.SemaphoreType.DMA,
      ],
  )
  return fn(src, dst, hpad, emb, zv)


def _sc_scatter_body(xs_h, src_h, dst_h, zz_h, out_h,
                     agg_sh, sidx_v, didx_v, rows_v, sem):
  c = lax.axis_index("c")
  s = lax.axis_index("s")

  pltpu.sync_copy(zz_h, agg_sh.at[pl.ds(s * SEG, SEG)])
  plsc.subcore_barrier()

  eb = (c * NS + s) * EPT

  def edge_batch(b, carry):
    base = eb + b * KB
    pltpu.sync_copy(src_h.at[pl.ds(base, KB)], sidx_v)
    pltpu.sync_copy(dst_h.at[pl.ds(base, KB)], didx_v)
    pltpu.async_copy(xs_h.at[sidx_v], rows_v, sem).wait()
    pltpu.sync_copy(rows_v, agg_sh.at[didx_v], add=True)
    return carry
  lax.fori_loop(0, NB, edge_batch, 0)

  plsc.subcore_barrier()
  pltpu.sync_copy(agg_sh.at[pl.ds(s * SEG, SEG)],
                  out_h.at[c, pl.ds(s * SEG, SEG)])


def _sc_scatter(xs, src, dst, zz):
  fn = pl.kernel(
      _sc_scatter_body,
      out_type=jax.ShapeDtypeStruct((NC, NP, H), _F32),
      mesh=_sc_mesh(),
      scratch_types=[
          pltpu.VMEM_SHARED((NP, H), _F32),
          pltpu.VMEM((KB,), jnp.int32),
          pltpu.VMEM((KB,), jnp.int32),
          pltpu.VMEM((KB, H), _F32),
          pltpu.SemaphoreType.DMA,
      ],
  )
  return fn(xs, src, dst, zz)


def _prep_body(dpt_ref, x0_ref, norms_ref, xs_ref):
  d = jnp.sum(dpt_ref[...], axis=-1)
  nrm = jnp.where(d > 0, lax.rsqrt(jnp.maximum(d, 1.0)), 0.0)
  norms_ref[...] = nrm
  xs_ref[...] = x0_ref[...] * nrm[:, 0:1]


def _prep(dpt, x0):
  bm = 2048
  return pl.pallas_call(
      _prep_body,
      grid=(NP // bm,),
      in_specs=[
          pl.BlockSpec((bm, 2, NC), lambda i: (i, 0, 0)),
          pl.BlockSpec((bm, H), lambda i: (i, 0)),
      ],
      out_specs=[
          pl.BlockSpec((bm, 2), lambda i: (i, 0)),
          pl.BlockSpec((bm, H), lambda i: (i, 0)),
      ],
      out_shape=[
          jax.ShapeDtypeStruct((NP, 2), _F32),
          jax.ShapeDtypeStruct((NP, H), _F32),
      ],
  )(dpt, x0)


def _layer_body(agg0_ref, agg1_ref, norms_ref, x_ref, w_ref, b_ref,
                g_ref, be_ref, xo_ref, xso_ref):
  agg = (agg0_ref[...] + agg1_ref[...]) * norms_ref[:, 1:2]
  y = jnp.dot(agg, w_ref[...], preferred_element_type=_F32) + b_ref[...]
  xo = x_ref[...] + jnp.maximum(g_ref[...] * y + be_ref[...], 0.0)
  xo_ref[...] = xo
  xso_ref[...] = xo * norms_ref[:, 0:1]


def _layer(agg0, agg1, norms, x, w, b, g, be):
  bm = 1024
  return pl.pallas_call(
      _layer_body,
      grid=(NP // bm,),
      in_specs=[
          pl.BlockSpec((bm, H), lambda i: (i, 0)),
          pl.BlockSpec((bm, H), lambda i: (i, 0)),
          pl.BlockSpec((bm, 2), lambda i: (i, 0)),
          pl.BlockSpec((bm, H), lambda i: (i, 0)),
          pl.BlockSpec((H, H), lambda i: (0, 0)),
          pl.BlockSpec((1, H), lambda i: (0, 0)),
          pl.BlockSpec((1, H), lambda i: (0, 0)),
          pl.BlockSpec((1, H), lambda i: (0, 0)),
      ],
      out_specs=[
          pl.BlockSpec((bm, H), lambda i: (i, 0)),
          pl.BlockSpec((bm, H), lambda i: (i, 0)),
      ],
      out_shape=[
          jax.ShapeDtypeStruct((NP, H), _F32),
          jax.ShapeDtypeStruct((NP, H), _F32),
      ],
  )(agg0, agg1, norms, x, w, b, g, be)


def _readout_body(x_ref, w1_ref, b1_ref, w2_ref, b2_ref, w3_ref, b3_ref,
                  out_ref, acc_ref):
  i = pl.program_id(0)

  @pl.when(i == 0)
  def _():
    acc_ref[...] = jnp.zeros_like(acc_ref)

  acc_ref[...] += jnp.sum(x_ref[...], axis=0, keepdims=True)

  @pl.when(i == pl.num_programs(0) - 1)
  def _():
    hg = acc_ref[...] * (1.0 / N)
    y = jnp.dot(hg, w1_ref[...], preferred_element_type=_F32) + b1_ref[...]
    y = jnp.maximum(y, 0.0)
    y = jnp.dot(y, w2_ref[...], preferred_element_type=_F32) + b2_ref[...]
    y = jnp.maximum(y, 0.0)
    out_ref[...] = (jnp.dot(y, w3_ref[...], preferred_element_type=_F32)
                    + b3_ref[...])


def _readout(x, w1, b1, w2, b2, w3, b3):
  bm = 400
  return pl.pallas_call(
      _readout_body,
      grid=(N // bm,),
      in_specs=[
          pl.BlockSpec((bm, H), lambda i: (i, 0)),
          pl.BlockSpec((H, H // 2), lambda i: (0, 0)),
          pl.BlockSpec((1, H // 2), lambda i: (0, 0)),
          pl.BlockSpec((H // 2, H // 4), lambda i: (0, 0)),
          pl.BlockSpec((1, H // 4), lambda i: (0, 0)),
          pl.BlockSpec((H // 4, 1), lambda i: (0, 0)),
          pl.BlockSpec((1, 1), lambda i: (0, 0)),
      ],
      out_specs=pl.BlockSpec((1, 1), lambda i: (0, 0)),
      out_shape=jax.ShapeDtypeStruct((1, 1), _F32),
      scratch_shapes=[pltpu.VMEM((1, H), _F32)],
  )(x, w1, b1, w2, b2, w3, b3)


def kernel(h, edge_index, e, emb, Ws, bs, gammas, betas,
           mlpW1, mlpb1, mlpW2, mlpb2, mlpW3, mlpb3):
  src = edge_index[0]
  dst = edge_index[1]
  hpad = jnp.concatenate([h, jnp.zeros((NP - N,), jnp.int32)])
  zz = jnp.zeros((SEG, H), _F32)
  zv = jnp.zeros((SEG,), _F32)

  deg_part, x0 = _sc_deg_emb(src, dst, hpad, emb, zv)
  dpt = jnp.transpose(deg_part, (2, 1, 0))
  norms, xs = _prep(dpt, x0)

  x = x0
  for i in range(L):
    aggs = _sc_scatter(xs, src, dst, zz)
    x, xs = _layer(aggs[0], aggs[1], norms, x,
                   Ws[i], bs[i].reshape(1, H),
                   gammas[i].reshape(1, H), betas[i].reshape(1, H))

  return _readout(x, mlpW1, mlpb1.reshape(1, H // 2),
                  mlpW2, mlpb2.reshape(1, H // 4),
                  mlpW3, mlpb3.reshape(1, 1))


# speedup vs baseline: 8.1153x; 1.9501x over previous
import jax
import jax.numpy as jnp
from jax import lax
from jax.experimental import pallas as pl
from jax.experimental.pallas import tpu as pltpu
from jax.experimental.pallas import tpu_sc as plsc

N = 10000
NP = 10240
E = 320000
H = 128
L = 4

NC = 2
NS = 16
NW = NC * NS
EPT = E // NW
KB = 80
NB = EPT // KB
SEG = NP // NS
GB = NP // (KB * NW)

_F32 = jnp.float32


def _sc_mesh():
  return plsc.VectorSubcoreMesh(core_axis_name="c", subcore_axis_name="s",
                                num_cores=NC, num_subcores=NS)


def _sc_deg_emb_body(src_h, dst_h, hpad_h, emb_h, zv_h, deg_h, x0_h,
                     dego_sh, degi_sh, ones_v, idx_v, rows_v, sem):
  c = lax.axis_index("c")
  s = lax.axis_index("s")
  wid = c * NS + s

  pltpu.sync_copy(zv_h, dego_sh.at[pl.ds(s * SEG, SEG)])
  pltpu.sync_copy(zv_h, degi_sh.at[pl.ds(s * SEG, SEG)])

  def fill_ones(i, carry):
    ones_v[pl.ds(i * 16, 16)] = jnp.ones((16,), _F32)
    return carry
  lax.fori_loop(0, KB // 16, fill_ones, 0)
  plsc.subcore_barrier()

  eb = (c * NS + s) * EPT

  def edge_batch(b, carry):
    base = eb + b * KB
    pltpu.sync_copy(src_h.at[pl.ds(base, KB)], idx_v)
    pltpu.sync_copy(ones_v, dego_sh.at[idx_v], add=True)
    pltpu.sync_copy(dst_h.at[pl.ds(base, KB)], idx_v)
    pltpu.sync_copy(ones_v, degi_sh.at[idx_v], add=True)
    return carry
  lax.fori_loop(0, NB, edge_batch, 0)

  plsc.subcore_barrier()
  pltpu.sync_copy(dego_sh.at[pl.ds(s * SEG, SEG)],
                  deg_h.at[c, 0, pl.ds(s * SEG, SEG)])
  pltpu.sync_copy(degi_sh.at[pl.ds(s * SEG, SEG)],
                  deg_h.at[c, 1, pl.ds(s * SEG, SEG)])

  for t in range(GB):
    j = wid + t * NW
    pltpu.sync_copy(hpad_h.at[pl.ds(j * KB, KB)], idx_v)
    pltpu.async_copy(emb_h.at[idx_v], rows_v, sem).wait()
    pltpu.sync_copy(rows_v, x0_h.at[pl.ds(j * KB, KB)])


def _sc_deg_emb(src, dst, hpad, emb, zv):
  fn = pl.kernel(
      _sc_deg_emb_body,
      out_type=[
          jax.ShapeDtypeStruct((NC, 2, NP), _F32),
          jax.ShapeDtypeStruct((NP, H), _F32),
      ],
      mesh=_sc_mesh(),
      scratch_types=[
          pltpu.VMEM_SHARED((NP,), _F32),
          pltpu.VMEM_SHARED((NP,), _F32),
          pltpu.VMEM((KB,), _F32),
          pltpu.VMEM((KB,), jnp.int32),
          pltpu.VMEM((KB, H), _F32),
          pltpu.SemaphoreType.DMA,
      ],
  )
  return fn(src, dst, hpad, emb, zv)


def _sc_scatter_body(xs_h, src_h, dst2_h, zz_h, out_h,
                     agg_sh, sidx_v, didx_v, rows0_v, rows1_v,
                     gsem0, gsem1, ssem0, ssem1):
  c = lax.axis_index("c")
  s = lax.axis_index("s")
  wid = c * NS + s
  eb = wid * EPT

  pltpu.sync_copy(zz_h, agg_sh.at[pl.ds(s * SEG, SEG)])
  pltpu.sync_copy(src_h.at[pl.ds(eb, EPT)], sidx_v)
  pltpu.sync_copy(dst2_h.at[pl.ds(eb, EPT)], didx_v)
  plsc.subcore_barrier()

  def start_gather(j, rbuf, sem):
    pltpu.async_copy(xs_h.at[sidx_v.at[pl.ds(j * KB, KB)]], rbuf, sem)

  def start_scatter(j, rbuf, sem):
    pltpu.async_copy(rbuf, agg_sh.at[didx_v.at[pl.ds(j * KB, KB)]], sem,
                     add=True)

  def wait_gather(rbuf, sem):
    pltpu.make_async_copy(xs_h.at[pl.ds(0, KB)], rbuf, sem).wait()

  def wait_scatter(rbuf, sem):
    pltpu.make_async_copy(rbuf, agg_sh.at[pl.ds(0, KB)], sem).wait()

  start_gather(0, rows0_v, gsem0)

  def pipe(g, carry):
    j0 = 2 * g

    @pl.when(g > 0)
    def _():
      wait_scatter(rows1_v, ssem1)
    start_gather(j0 + 1, rows1_v, gsem1)
    wait_gather(rows0_v, gsem0)
    start_scatter(j0, rows0_v, ssem0)

    wait_scatter(rows0_v, ssem0)
    start_gather(j0 + 2, rows0_v, gsem0)
    wait_gather(rows1_v, gsem1)
    start_scatter(j0 + 1, rows1_v, ssem1)
    return carry
  lax.fori_loop(0, (NB - 1) // 2, pipe, 0)

  wait_gather(rows0_v, gsem0)
  start_scatter(NB - 1, rows0_v, ssem0)
  wait_scatter(rows1_v, ssem1)
  wait_scatter(rows0_v, ssem0)

  plsc.subcore_barrier()
  pltpu.sync_copy(agg_sh.at[pl.ds(s * SEG, SEG)],
                  out_h.at[c, pl.ds(s * SEG, SEG)])


def _sc_scatter(xs, src, dst2, zz):
  fn = pl.kernel(
      _sc_scatter_body,
      out_type=jax.ShapeDtypeStruct((NC, NP, H), _F32),
      mesh=_sc_mesh(),
      scratch_types=[
          pltpu.VMEM_SHARED((NP, H), _F32),
          pltpu.VMEM((EPT,), jnp.int32),
          pltpu.VMEM((EPT,), jnp.int32),
          pltpu.VMEM((KB, H), _F32),
          pltpu.VMEM((KB, H), _F32),
          pltpu.SemaphoreType.DMA,
          pltpu.SemaphoreType.DMA,
          pltpu.SemaphoreType.DMA,
          pltpu.SemaphoreType.DMA,
      ],
  )
  return fn(xs, src, dst2, zz)


def _prep_body(dpt_ref, x0_ref, norms_ref, xs_ref):
  d = jnp.sum(dpt_ref[...], axis=-1)
  nrm = jnp.where(d > 0, lax.rsqrt(jnp.maximum(d, 1.0)), 0.0)
  norms_ref[...] = nrm
  xs_ref[...] = x0_ref[...] * nrm[:, 0:1]


def _prep(dpt, x0):
  bm = 2048
  return pl.pallas_call(
      _prep_body,
      grid=(NP // bm,),
      in_specs=[
          pl.BlockSpec((bm, 2, NC), lambda i: (i, 0, 0)),
          pl.BlockSpec((bm, H), lambda i: (i, 0)),
      ],
      out_specs=[
          pl.BlockSpec((bm, 2), lambda i: (i, 0)),
          pl.BlockSpec((bm, H), lambda i: (i, 0)),
      ],
      out_shape=[
          jax.ShapeDtypeStruct((NP, 2), _F32),
          jax.ShapeDtypeStruct((NP, H), _F32),
      ],
  )(dpt, x0)


def _layer_body(agg0_ref, agg1_ref, norms_ref, x_ref, w_ref, b_ref,
                g_ref, be_ref, xo_ref, xso_ref):
  agg = (agg0_ref[...] + agg1_ref[...]) * norms_ref[:, 1:2]
  y = jnp.dot(agg, w_ref[...], preferred_element_type=_F32) + b_ref[...]
  xo = x_ref[...] + jnp.maximum(g_ref[...] * y + be_ref[...], 0.0)
  xo_ref[...] = xo
  xso_ref[...] = xo * norms_ref[:, 0:1]


def _layer(agg0, agg1, norms, x, w, b, g, be):
  bm = 1024
  return pl.pallas_call(
      _layer_body,
      grid=(NP // bm,),
      in_specs=[
          pl.BlockSpec((bm, H), lambda i: (i, 0)),
          pl.BlockSpec((bm, H), lambda i: (i, 0)),
          pl.BlockSpec((bm, 2), lambda i: (i, 0)),
          pl.BlockSpec((bm, H), lambda i: (i, 0)),
          pl.BlockSpec((H, H), lambda i: (0, 0)),
          pl.BlockSpec((1, H), lambda i: (0, 0)),
          pl.BlockSpec((1, H), lambda i: (0, 0)),
          pl.BlockSpec((1, H), lambda i: (0, 0)),
      ],
      out_specs=[
          pl.BlockSpec((bm, H), lambda i: (i, 0)),
          pl.BlockSpec((bm, H), lambda i: (i, 0)),
      ],
      out_shape=[
          jax.ShapeDtypeStruct((NP, H), _F32),
          jax.ShapeDtypeStruct((NP, H), _F32),
      ],
  )(agg0, agg1, norms, x, w, b, g, be)


def _readout_body(x_ref, w1_ref, b1_ref, w2_ref, b2_ref, w3_ref, b3_ref,
                  out_ref, acc_ref):
  i = pl.program_id(0)

  @pl.when(i == 0)
  def _():
    acc_ref[...] = jnp.zeros_like(acc_ref)

  acc_ref[...] += jnp.sum(x_ref[...], axis=0, keepdims=True)

  @pl.when(i == pl.num_programs(0) - 1)
  def _():
    hg = acc_ref[...] * (1.0 / N)
    y = jnp.dot(hg, w1_ref[...], preferred_element_type=_F32) + b1_ref[...]
    y = jnp.maximum(y, 0.0)
    y = jnp.dot(y, w2_ref[...], preferred_element_type=_F32) + b2_ref[...]
    y = jnp.maximum(y, 0.0)
    out_ref[...] = (jnp.dot(y, w3_ref[...], preferred_element_type=_F32)
                    + b3_ref[...])


def _readout(x, w1, b1, w2, b2, w3, b3):
  bm = 400
  return pl.pallas_call(
      _readout_body,
      grid=(N // bm,),
      in_specs=[
          pl.BlockSpec((bm, H), lambda i: (i, 0)),
          pl.BlockSpec((H, H // 2), lambda i: (0, 0)),
          pl.BlockSpec((1, H // 2), lambda i: (0, 0)),
          pl.BlockSpec((H // 2, H // 4), lambda i: (0, 0)),
          pl.BlockSpec((1, H // 4), lambda i: (0, 0)),
          pl.BlockSpec((H // 4, 1), lambda i: (0, 0)),
          pl.BlockSpec((1, 1), lambda i: (0, 0)),
      ],
      out_specs=pl.BlockSpec((1, 1), lambda i: (0, 0)),
      out_shape=jax.ShapeDtypeStruct((1, 1), _F32),
      scratch_shapes=[pltpu.VMEM((1, H), _F32)],
  )(x, w1, b1, w2, b2, w3, b3)


def kernel(h, edge_index, e, emb, Ws, bs, gammas, betas,
           mlpW1, mlpb1, mlpW2, mlpb2, mlpW3, mlpb3):
  src = edge_index[0]
  dst = edge_index[1]
  hpad = jnp.concatenate([h, jnp.zeros((NP - N,), jnp.int32)])
  zz = jnp.zeros((SEG, H), _F32)
  zv = jnp.zeros((SEG,), _F32)

  deg_part, x0 = _sc_deg_emb(src, dst, hpad, emb, zv)
  dpt = jnp.transpose(deg_part, (2, 1, 0))
  norms, xs = _prep(dpt, x0)

  x = x0
  for i in range(L):
    aggs = _sc_scatter(xs, src, dst, zz)
    x, xs = _layer(aggs[0], aggs[1], norms, x,
                   Ws[i], bs[i].reshape(1, H),
                   gammas[i].reshape(1, H), betas[i].reshape(1, H))

  return _readout(x, mlpW1, mlpb1.reshape(1, H // 2),
                  mlpW2, mlpb2.reshape(1, H // 4),
                  mlpW3, mlpb3.reshape(1, 1))


# speedup vs baseline: 10.9075x; 1.3441x over previous
import jax
import jax.numpy as jnp
from jax import lax
from jax.experimental import pallas as pl
from jax.experimental.pallas import tpu as pltpu
from jax.experimental.pallas import tpu_sc as plsc

N = 10000
NP = 10240
E = 320000
H = 128
L = 4

NC = 2
NS = 16
NW = NC * NS
EPT = E // NW
KB = 80
NB = EPT // KB
SEG = NP // NS
GB = NP // (KB * NW)

_F32 = jnp.float32


def _sc_mesh():
  return plsc.VectorSubcoreMesh(core_axis_name="c", subcore_axis_name="s",
                                num_cores=NC, num_subcores=NS)


def _sc_deg_emb_body(src_h, dst_h, hpad_h, emb_h, zv_h, deg_h, x0_h,
                     dego_sh, degi_sh, ones_v, sidx_v, didx_v, hidx_v,
                     er0, er1, dsem, g0, g1):
  c = lax.axis_index("c")
  s = lax.axis_index("s")
  wid = c * NS + s
  eb = wid * EPT

  pltpu.sync_copy(zv_h, dego_sh.at[pl.ds(s * SEG, SEG)])
  pltpu.sync_copy(zv_h, degi_sh.at[pl.ds(s * SEG, SEG)])

  def fill_ones(i, carry):
    ones_v[pl.ds(i * 16, 16)] = jnp.ones((16,), _F32)
    return carry
  lax.fori_loop(0, KB // 16, fill_ones, 0)

  pltpu.sync_copy(src_h.at[pl.ds(eb, EPT)], sidx_v)
  pltpu.sync_copy(dst_h.at[pl.ds(eb, EPT)], didx_v)
  pltpu.sync_copy(hpad_h.at[pl.ds(wid * GB * KB, GB * KB)], hidx_v)
  plsc.subcore_barrier()

  ebufs = (er0, er1)
  esems = (g0, g1)
  xb = wid * GB * KB
  for t in range(GB):
    pltpu.async_copy(emb_h.at[hidx_v.at[pl.ds(t * KB, KB)]],
                     ebufs[t % 2], esems[t % 2])
    if t >= 1:
      pltpu.make_async_copy(emb_h.at[pl.ds(0, KB)], ebufs[(t - 1) % 2],
                            esems[(t - 1) % 2]).wait()
      pltpu.sync_copy(ebufs[(t - 1) % 2],
                      x0_h.at[pl.ds(xb + (t - 1) * KB, KB)])
  pltpu.make_async_copy(emb_h.at[pl.ds(0, KB)], ebufs[(GB - 1) % 2],
                        esems[(GB - 1) % 2]).wait()
  pltpu.sync_copy(ebufs[(GB - 1) % 2],
                  x0_h.at[pl.ds(xb + (GB - 1) * KB, KB)])

  GRP = 8

  def drain_deg(n):
    def w(i, carry):
      pltpu.make_async_copy(zv_h.at[pl.ds(0, KB)], ones_v, dsem).wait()
      return carry
    lax.fori_loop(0, n, w, 0)

  def dgroup(g, carry):
    for b in range(GRP):
      j = g * GRP + b
      pltpu.async_copy(ones_v, dego_sh.at[sidx_v.at[pl.ds(j * KB, KB)]],
                       dsem, add=True)
      pltpu.async_copy(ones_v, degi_sh.at[didx_v.at[pl.ds(j * KB, KB)]],
                       dsem, add=True)
    drain_deg(2 * GRP)
    return carry
  lax.fori_loop(0, NB // GRP, dgroup, 0)
  for j in range((NB // GRP) * GRP, NB):
    pltpu.async_copy(ones_v, dego_sh.at[sidx_v.at[pl.ds(j * KB, KB)]],
                     dsem, add=True)
    pltpu.async_copy(ones_v, degi_sh.at[didx_v.at[pl.ds(j * KB, KB)]],
                     dsem, add=True)
  drain_deg(2 * (NB - (NB // GRP) * GRP))

  plsc.subcore_barrier()
  pltpu.sync_copy(dego_sh.at[pl.ds(s * SEG, SEG)],
                  deg_h.at[c, 0, pl.ds(s * SEG, SEG)])
  pltpu.sync_copy(degi_sh.at[pl.ds(s * SEG, SEG)],
                  deg_h.at[c, 1, pl.ds(s * SEG, SEG)])


def _sc_deg_emb(src, dst, hpad, emb, zv):
  fn = pl.kernel(
      _sc_deg_emb_body,
      out_type=[
          jax.ShapeDtypeStruct((NC, 2, NP), _F32),
          jax.ShapeDtypeStruct((NP, H), _F32),
      ],
      mesh=_sc_mesh(),
      scratch_types=[
          pltpu.VMEM_SHARED((NP,), _F32),
          pltpu.VMEM_SHARED((NP,), _F32),
          pltpu.VMEM((KB,), _F32),
          pltpu.VMEM((EPT,), jnp.int32),
          pltpu.VMEM((EPT,), jnp.int32),
          pltpu.VMEM((GB * KB,), jnp.int32),
          pltpu.VMEM((KB, H), _F32),
          pltpu.VMEM((KB, H), _F32),
          pltpu.SemaphoreType.DMA,
          pltpu.SemaphoreType.DMA,
          pltpu.SemaphoreType.DMA,
      ],
  )
  return fn(src, dst, hpad, emb, zv)


_NBUF = 3
_CH = 60


def _sc_scatter_body(xs_h, src_h, dst_h, zz_h, out_h,
                     agg_sh, sidx_v, didx_v, r0, r1, r2,
                     g0, g1, g2, s0, s1, s2):
  c = lax.axis_index("c")
  s = lax.axis_index("s")
  wid = c * NS + s
  eb = wid * EPT
  rows = (r0, r1, r2)
  gs = (g0, g1, g2)
  ss = (s0, s1, s2)

  pltpu.sync_copy(zz_h, agg_sh.at[pl.ds(s * SEG, SEG)])
  plsc.subcore_barrier()

  def start_gather(jl, b):
    pltpu.async_copy(xs_h.at[sidx_v.at[pl.ds(jl * KB, KB)]], rows[b], gs[b])

  def start_scatter(jl, b):
    pltpu.async_copy(rows[b], agg_sh.at[didx_v.at[pl.ds(jl * KB, KB)]],
                     ss[b], add=True)

  def wait_gather(b):
    pltpu.make_async_copy(xs_h.at[pl.ds(0, KB)], rows[b], gs[b]).wait()

  def wait_scatter(b):
    pltpu.make_async_copy(rows[b], agg_sh.at[pl.ds(0, KB)], ss[b]).wait()

  for cbase, cnt in ((0, _CH), (_CH, _CH), (2 * _CH, NB - 2 * _CH)):
    base = eb + cbase * KB
    pltpu.sync_copy(src_h.at[pl.ds(base, cnt * KB)],
                    sidx_v.at[pl.ds(0, cnt * KB)])
    pltpu.sync_copy(dst_h.at[pl.ds(base, cnt * KB)],
                    didx_v.at[pl.ds(0, cnt * KB)])

    for b in range(_NBUF):
      start_gather(b, b)

    def pipe(g, carry, cnt=cnt):
      for b in range(_NBUF):
        jl = _NBUF * g + b
        wait_gather(b)
        start_scatter(jl, b)
        wait_scatter(b)

        @pl.when(jl + _NBUF < cnt)
        def _():
          start_gather(jl + _NBUF, b)
      return carry
    lax.fori_loop(0, cnt // _NBUF, pipe, 0)

    for jl in range((cnt // _NBUF) * _NBUF, cnt):
      b = jl % _NBUF
      wait_gather(b)
      start_scatter(jl, b)
      wait_scatter(b)

  plsc.subcore_barrier()
  pltpu.sync_copy(agg_sh.at[pl.ds(s * SEG, SEG)],
                  out_h.at[c, pl.ds(s * SEG, SEG)])


def _sc_scatter(xs, src, dst, zz):
  fn = pl.kernel(
      _sc_scatter_body,
      out_type=jax.ShapeDtypeStruct((NC, NP, H), _F32),
      mesh=_sc_mesh(),
      scratch_types=[
          pltpu.VMEM_SHARED((NP, H), _F32),
          pltpu.VMEM((_CH * KB,), jnp.int32),
          pltpu.VMEM((_CH * KB,), jnp.int32),
          pltpu.VMEM((KB, H), _F32),
          pltpu.VMEM((KB, H), _F32),
          pltpu.VMEM((KB, H), _F32),
          pltpu.SemaphoreType.DMA,
          pltpu.SemaphoreType.DMA,
          pltpu.SemaphoreType.DMA,
          pltpu.SemaphoreType.DMA,
          pltpu.SemaphoreType.DMA,
          pltpu.SemaphoreType.DMA,
      ],
  )
  return fn(xs, src, dst, zz)


def _prep_body(dpt_ref, x0_ref, norms_ref, xs_ref):
  d = jnp.sum(dpt_ref[...], axis=-1)
  nrm = jnp.where(d > 0, lax.rsqrt(jnp.maximum(d, 1.0)), 0.0)
  norms_ref[...] = nrm
  xs_ref[...] = x0_ref[...] * nrm[:, 0:1]


def _prep(dpt, x0):
  bm = 2048
  return pl.pallas_call(
      _prep_body,
      grid=(NP // bm,),
      in_specs=[
          pl.BlockSpec((bm, 2, NC), lambda i: (i, 0, 0)),
          pl.BlockSpec((bm, H), lambda i: (i, 0)),
      ],
      out_specs=[
          pl.BlockSpec((bm, 2), lambda i: (i, 0)),
          pl.BlockSpec((bm, H), lambda i: (i, 0)),
      ],
      out_shape=[
          jax.ShapeDtypeStruct((NP, 2), _F32),
          jax.ShapeDtypeStruct((NP, H), _F32),
      ],
  )(dpt, x0)


def _layer_body(agg0_ref, agg1_ref, norms_ref, x_ref, w_ref, b_ref,
                g_ref, be_ref, xo_ref, xso_ref):
  agg = (agg0_ref[...] + agg1_ref[...]) * norms_ref[:, 1:2]
  y = jnp.dot(agg, w_ref[...], preferred_element_type=_F32) + b_ref[...]
  xo = x_ref[...] + jnp.maximum(g_ref[...] * y + be_ref[...], 0.0)
  xo_ref[...] = xo
  xso_ref[...] = xo * norms_ref[:, 0:1]


def _layer(agg0, agg1, norms, x, w, b, g, be):
  bm = 1024
  return pl.pallas_call(
      _layer_body,
      grid=(NP // bm,),
      in_specs=[
          pl.BlockSpec((bm, H), lambda i: (i, 0)),
          pl.BlockSpec((bm, H), lambda i: (i, 0)),
          pl.BlockSpec((bm, 2), lambda i: (i, 0)),
          pl.BlockSpec((bm, H), lambda i: (i, 0)),
          pl.BlockSpec((H, H), lambda i: (0, 0)),
          pl.BlockSpec((1, H), lambda i: (0, 0)),
          pl.BlockSpec((1, H), lambda i: (0, 0)),
          pl.BlockSpec((1, H), lambda i: (0, 0)),
      ],
      out_specs=[
          pl.BlockSpec((bm, H), lambda i: (i, 0)),
          pl.BlockSpec((bm, H), lambda i: (i, 0)),
      ],
      out_shape=[
          jax.ShapeDtypeStruct((NP, H), _F32),
          jax.ShapeDtypeStruct((NP, H), _F32),
      ],
  )(agg0, agg1, norms, x, w, b, g, be)


_BM = 1024


def _layer_last_body(agg0_ref, agg1_ref, norms_ref, x_ref, w_ref, b_ref,
                     g_ref, be_ref, w1_ref, b1_ref, w2_ref, b2_ref,
                     w3_ref, b3_ref, out_ref, acc_ref):
  i = pl.program_id(0)

  @pl.when(i == 0)
  def _():
    acc_ref[...] = jnp.zeros_like(acc_ref)

  agg = (agg0_ref[...] + agg1_ref[...]) * norms_ref[:, 1:2]
  y = jnp.dot(agg, w_ref[...], preferred_element_type=_F32) + b_ref[...]
  xo = x_ref[...] + jnp.maximum(g_ref[...] * y + be_ref[...], 0.0)
  rowid = i * _BM + lax.broadcasted_iota(jnp.int32, (_BM, 1), 0)
  xo = jnp.where(rowid < N, xo, 0.0)
  acc_ref[...] += jnp.sum(xo, axis=0, keepdims=True)

  @pl.when(i == pl.num_programs(0) - 1)
  def _():
    hg = acc_ref[...] * (1.0 / N)
    z = jnp.dot(hg, w1_ref[...], preferred_element_type=_F32) + b1_ref[...]
    z = jnp.maximum(z, 0.0)
    z = jnp.dot(z, w2_ref[...], preferred_element_type=_F32) + b2_ref[...]
    z = jnp.maximum(z, 0.0)
    out_ref[...] = (jnp.dot(z, w3_ref[...], preferred_element_type=_F32)
                    + b3_ref[...])


def _layer_last(agg0, agg1, norms, x, w, b, g, be, w1, b1, w2, b2, w3, b3):
  bm = _BM
  return pl.pallas_call(
      _layer_last_body,
      grid=(NP // bm,),
      in_specs=[
          pl.BlockSpec((bm, H), lambda i: (i, 0)),
          pl.BlockSpec((bm, H), lambda i: (i, 0)),
          pl.BlockSpec((bm, 2), lambda i: (i, 0)),
          pl.BlockSpec((bm, H), lambda i: (i, 0)),
          pl.BlockSpec((H, H), lambda i: (0, 0)),
          pl.BlockSpec((1, H), lambda i: (0, 0)),
          pl.BlockSpec((1, H), lambda i: (0, 0)),
          pl.BlockSpec((1, H), lambda i: (0, 0)),
          pl.BlockSpec((H, H // 2), lambda i: (0, 0)),
          pl.BlockSpec((1, H // 2), lambda i: (0, 0)),
          pl.BlockSpec((H // 2, H // 4), lambda i: (0, 0)),
          pl.BlockSpec((1, H // 4), lambda i: (0, 0)),
          pl.BlockSpec((H // 4, 1), lambda i: (0, 0)),
          pl.BlockSpec((1, 1), lambda i: (0, 0)),
      ],
      out_specs=pl.BlockSpec((1, 1), lambda i: (0, 0)),
      out_shape=jax.ShapeDtypeStruct((1, 1), _F32),
      scratch_shapes=[pltpu.VMEM((1, H), _F32)],
  )(agg0, agg1, norms, x, w, b, g, be, w1, b1, w2, b2, w3, b3)


def kernel(h, edge_index, e, emb, Ws, bs, gammas, betas,
           mlpW1, mlpb1, mlpW2, mlpb2, mlpW3, mlpb3):
  src = edge_index[0]
  dst = edge_index[1]
  hpad = jnp.concatenate([h, jnp.zeros((NP - N,), jnp.int32)])
  zz = jnp.zeros((SEG, H), _F32)
  zv = jnp.zeros((SEG,), _F32)

  deg_part, x0 = _sc_deg_emb(src, dst, hpad, emb, zv)
  dpt = jnp.transpose(deg_part, (2, 1, 0))
  norms, xs = _prep(dpt, x0)

  x = x0
  for i in range(L - 1):
    aggs = _sc_scatter(xs, src, dst, zz)
    x, xs = _layer(aggs[0], aggs[1], norms, x,
                   Ws[i], bs[i].reshape(1, H),
                   gammas[i].reshape(1, H), betas[i].reshape(1, H))

  aggs = _sc_scatter(xs, src, dst, zz)
  return _layer_last(aggs[0], aggs[1], norms, x,
                     Ws[L - 1], bs[L - 1].reshape(1, H),
                     gammas[L - 1].reshape(1, H), betas[L - 1].reshape(1, H),
                     mlpW1, mlpb1.reshape(1, H // 2),
                     mlpW2, mlpb2.reshape(1, H // 4),
                     mlpW3, mlpb3.reshape(1, 1))
